# baseline (speedup 1.0000x reference)
import functools

import jax
import jax.numpy as jnp
from jax.experimental import pallas as pl
from jax.experimental.pallas import tpu as pltpu

H = 4096
A = 30


def _sample_body(pos_ref, trails_row, gumbel_row, paths_row,
                 next_out, plen2_out):
    i = pl.program_id(0)
    score = trails_row[0, :, :] + gumbel_row[0, :, :]
    m = jnp.max(score)
    col = jax.lax.broadcasted_iota(jnp.int32, (1, H), 1)
    nxt = jnp.min(jnp.where(score == m, col, jnp.int32(H)))
    ap = paths_row[0, :, :]
    apn = jnp.sum(jnp.where(col == nxt, ap, 0.0))
    plen2 = jnp.sum(ap * ap) - apn * apn + 1.0
    next_out[i] = nxt
    plen2_out[i] = plen2


def _select_body(next_ref, plen2_ref, paths_ref, best_path_ref, blen_ref,
                 mask_out):
    def body(a, carry):
        bv, bi = carry
        v = plen2_ref[a]
        take = v < bv
        return jnp.where(take, v, bv), jnp.where(take, a, bi)

    bv, best = jax.lax.fori_loop(0, A, body, (jnp.float32(jnp.inf),
                                              jnp.int32(0)))
    best_len = jnp.sqrt(bv)
    better = best_len < blen_ref[0]
    row_id = jax.lax.broadcasted_iota(jnp.int32, (A, 1), 0)
    best_row = jnp.sum(jnp.where(row_id == best, paths_ref[:, :], 0.0),
                       axis=0, keepdims=True)
    nxt = next_ref[best]
    col = jax.lax.broadcasted_iota(jnp.int32, (1, H), 1)
    new_row = jnp.where(col == nxt, 1.0, best_row)
    mask_out[:, :] = jnp.where(better, new_row, best_path_ref[:, :])


def _mul_body(x_blk, mask_blk, out_blk):
    out_blk[:, :] = x_blk[:, :] * mask_blk[:, :]


def kernel(x, pheromone_trails, ant_positions, ant_paths, best_path,
           best_path_length, pheromone_decay, pheromone_strength):
    del pheromone_decay, pheromone_strength
    gumbel = jax.random.gumbel(jax.random.key(42), (A, H), jnp.float32)

    next_pos, plen2 = pl.pallas_call(
        _sample_body,
        grid_spec=pltpu.PrefetchScalarGridSpec(
            num_scalar_prefetch=1,
            grid=(A,),
            in_specs=[
                pl.BlockSpec((1, 1, H), lambda i, pos: (pos[i], 0, 0)),
                pl.BlockSpec((1, 1, H), lambda i, pos: (i, 0, 0)),
                pl.BlockSpec((1, 1, H), lambda i, pos: (i, 0, 0)),
            ],
            out_specs=[
                pl.BlockSpec(memory_space=pltpu.SMEM),
                pl.BlockSpec(memory_space=pltpu.SMEM),
            ],
        ),
        out_shape=[
            jax.ShapeDtypeStruct((A,), jnp.int32),
            jax.ShapeDtypeStruct((A,), jnp.float32),
        ],
    )(ant_positions, pheromone_trails.reshape(H, 1, H),
      gumbel.reshape(A, 1, H), ant_paths.reshape(A, 1, H))

    mask = pl.pallas_call(
        _select_body,
        in_specs=[
            pl.BlockSpec(memory_space=pltpu.SMEM),
            pl.BlockSpec(memory_space=pltpu.SMEM),
            pl.BlockSpec(memory_space=pltpu.VMEM),
            pl.BlockSpec(memory_space=pltpu.VMEM),
            pl.BlockSpec(memory_space=pltpu.SMEM),
        ],
        out_shape=jax.ShapeDtypeStruct((1, H), jnp.float32),
    )(next_pos, plen2, ant_paths, best_path.reshape(1, H),
      best_path_length.reshape(1))

    B, S, _ = x.shape
    R = B * S
    BR = 256
    x2 = x.reshape(R, H)
    out = pl.pallas_call(
        _mul_body,
        grid=(R // BR,),
        in_specs=[
            pl.BlockSpec((BR, H), lambda i: (i, 0)),
            pl.BlockSpec((1, H), lambda i: (0, 0)),
        ],
        out_specs=pl.BlockSpec((BR, H), lambda i: (i, 0)),
        out_shape=jax.ShapeDtypeStruct((R, H), jnp.float32),
    )(x2, mask)
    return out.reshape(B, S, H)
